# Initial kernel scaffold; baseline (speedup 1.0000x reference)
import jax
import jax.numpy as jnp
from jax.experimental import pallas as pl


def kernel(users, items, user_table, item_table):
    raise NotImplementedError("write your pallas kernel here")


# speedup vs baseline: 1.2188x; 1.2188x over previous
import functools

import jax
import jax.numpy as jnp
from jax import lax
from jax.experimental import pallas as pl
from jax.experimental.pallas import tpu as pltpu
from jax.experimental.pallas import tpu_sc as plsc

L = 16
NUM_WORKERS = 32
CHUNK = 128


def _gmf_body(users_hbm, items_hbm, ut_hbm, it_hbm, out_hbm,
              idx_u, idx_v, rows_u, rows_v, sem_u, sem_v):
  n_chunks = idx_u.shape[0]
  d = ut_hbm.shape[1]
  wid = lax.axis_index("s") * 2 + lax.axis_index("c")

  pltpu.sync_copy(users_hbm.at[wid], idx_u)
  pltpu.sync_copy(items_hbm.at[wid], idx_v)

  def fire(j, slot):
    cu = pltpu.async_copy(ut_hbm.at[idx_u.at[j]], rows_u.at[slot], sem_u)
    cv = pltpu.async_copy(it_hbm.at[idx_v.at[j]], rows_v.at[slot], sem_v)
    return cu, cv

  def drain_compute_store(j, slot, copies):
    cu, cv = copies
    cu.wait()
    cv.wait()

    def mul_row(r, _):
      for k in range(d // L):
        s = pl.ds(k * L, L)
        rows_u[slot, r, s] = rows_u[slot, r, s] * rows_v[slot, r, s]
      return _

    lax.fori_loop(0, CHUNK, mul_row, 0, unroll=2)
    pltpu.sync_copy(rows_u.at[slot],
                    out_hbm.at[pl.ds((wid * n_chunks + j) * CHUNK, CHUNK)])

  copies = fire(0, 0)
  for j in range(n_chunks):
    nxt = None
    if j + 1 < n_chunks:
      nxt = fire(j + 1, (j + 1) % 2)
    drain_compute_store(j, j % 2, copies)
    copies = nxt


def kernel(users, items, user_table, item_table):
  b = users.shape[0]
  d = user_table.shape[1]
  n_chunks = b // (NUM_WORKERS * CHUNK)

  users_r = users.astype(jnp.int32).reshape(NUM_WORKERS, n_chunks, CHUNK)
  items_r = items.astype(jnp.int32).reshape(NUM_WORKERS, n_chunks, CHUNK)

  mesh = plsc.VectorSubcoreMesh(core_axis_name="c", subcore_axis_name="s")
  run = functools.partial(
      pl.kernel,
      mesh=mesh,
      out_type=jax.ShapeDtypeStruct((b, d), jnp.float32),
      scratch_types=[
          pltpu.VMEM((n_chunks, CHUNK), jnp.int32),
          pltpu.VMEM((n_chunks, CHUNK), jnp.int32),
          pltpu.VMEM((2, CHUNK, d), jnp.float32),
          pltpu.VMEM((2, CHUNK, d), jnp.float32),
          pltpu.SemaphoreType.DMA,
          pltpu.SemaphoreType.DMA,
      ],
  )(_gmf_body)
  return run(users_r, items_r, user_table, item_table)


# speedup vs baseline: 1.2640x; 1.0371x over previous
import functools

import jax
import jax.numpy as jnp
from jax import lax
from jax.experimental import pallas as pl
from jax.experimental.pallas import tpu as pltpu
from jax.experimental.pallas import tpu_sc as plsc

L = 16
NUM_WORKERS = 32
CHUNK = 64
RING = 6


def _gmf_body(idx_hbm, ut_hbm, it_hbm, out_hbm,
              idx, rows_u, rows_v, sem_u, sem_v, sem_o):
  n_chunks = idx.shape[1]
  d = ut_hbm.shape[1]
  wid = lax.axis_index("s") * 2 + lax.axis_index("c")

  pltpu.sync_copy(idx_hbm.at[wid], idx)

  def fire(j):
    slot = j % RING
    cu = pltpu.async_copy(ut_hbm.at[idx.at[0, j]], rows_u.at[slot], sem_u)
    cv = pltpu.async_copy(it_hbm.at[idx.at[1, j]], rows_v.at[slot], sem_v)
    return cu, cv

  gathers = [fire(j) for j in range(min(RING, n_chunks))]
  gathers += [None] * (n_chunks - len(gathers))
  outs = [None] * n_chunks

  for j in range(n_chunks):
    if RING <= n_chunks and 1 <= j <= n_chunks - RING:
      outs[j - 1].wait()
      outs[j - 1] = None
      gathers[j + RING - 1] = fire(j + RING - 1)

    cu, cv = gathers[j]
    cu.wait()
    cv.wait()
    slot = j % RING

    def mul_row(r, _):
      for k in range(d // L):
        s = pl.ds(k * L, L)
        rows_u[slot, r, s] = rows_u[slot, r, s] * rows_v[slot, r, s]
      return _

    lax.fori_loop(0, CHUNK, mul_row, 0, unroll=4)
    outs[j] = pltpu.async_copy(
        rows_u.at[slot],
        out_hbm.at[pl.ds((wid * n_chunks + j) * CHUNK, CHUNK)], sem_o)

  for c in outs:
    if c is not None:
      c.wait()


def kernel(users, items, user_table, item_table):
  b = users.shape[0]
  d = user_table.shape[1]
  n_chunks = b // (NUM_WORKERS * CHUNK)

  idx = jnp.stack(
      [users.astype(jnp.int32).reshape(NUM_WORKERS, n_chunks, CHUNK),
       items.astype(jnp.int32).reshape(NUM_WORKERS, n_chunks, CHUNK)],
      axis=1)

  mesh = plsc.VectorSubcoreMesh(core_axis_name="c", subcore_axis_name="s")
  run = functools.partial(
      pl.kernel,
      mesh=mesh,
      out_type=jax.ShapeDtypeStruct((b, d), jnp.float32),
      scratch_types=[
          pltpu.VMEM((2, n_chunks, CHUNK), jnp.int32),
          pltpu.VMEM((RING, CHUNK, d), jnp.float32),
          pltpu.VMEM((RING, CHUNK, d), jnp.float32),
          pltpu.SemaphoreType.DMA,
          pltpu.SemaphoreType.DMA,
          pltpu.SemaphoreType.DMA,
      ],
  )(_gmf_body)
  return run(idx, user_table, item_table)


# speedup vs baseline: 1.2688x; 1.0038x over previous
import functools

import jax
import jax.numpy as jnp
from jax import lax
from jax.experimental import pallas as pl
from jax.experimental.pallas import tpu as pltpu
from jax.experimental.pallas import tpu_sc as plsc

L = 16
NUM_WORKERS = 32
CHUNK = 64
RING = 6


def _gmf_body(users_hbm, items_hbm, ut_hbm, it_hbm, out_hbm,
              idx_u, idx_v, rows_u, rows_v, sem_u, sem_v, sem_o):
  n_chunks = idx_u.shape[0] // CHUNK
  d = ut_hbm.shape[1]
  wid = lax.axis_index("s") * 2 + lax.axis_index("c")
  base = wid * n_chunks * CHUNK

  ci = pltpu.async_copy(users_hbm.at[pl.ds(base, n_chunks * CHUNK)], idx_u,
                        sem_u)
  cj = pltpu.async_copy(items_hbm.at[pl.ds(base, n_chunks * CHUNK)], idx_v,
                        sem_v)
  ci.wait()
  cj.wait()

  def fire(j):
    slot = j % RING
    cu = pltpu.async_copy(ut_hbm.at[idx_u.at[pl.ds(j * CHUNK, CHUNK)]],
                          rows_u.at[slot], sem_u)
    cv = pltpu.async_copy(it_hbm.at[idx_v.at[pl.ds(j * CHUNK, CHUNK)]],
                          rows_v.at[slot], sem_v)
    return cu, cv

  gathers = [fire(j) for j in range(min(RING, n_chunks))]
  gathers += [None] * (n_chunks - len(gathers))
  outs = [None] * n_chunks

  for j in range(n_chunks):
    if RING <= n_chunks and 1 <= j <= n_chunks - RING:
      outs[j - 1].wait()
      outs[j - 1] = None
      gathers[j + RING - 1] = fire(j + RING - 1)

    cu, cv = gathers[j]
    cu.wait()
    cv.wait()
    slot = j % RING

    def mul_row(r, _):
      for k in range(d // L):
        s = pl.ds(k * L, L)
        rows_u[slot, r, s] = rows_u[slot, r, s] * rows_v[slot, r, s]
      return _

    lax.fori_loop(0, CHUNK, mul_row, 0, unroll=4)
    outs[j] = pltpu.async_copy(
        rows_u.at[slot],
        out_hbm.at[pl.ds((wid * n_chunks + j) * CHUNK, CHUNK)], sem_o)

  for c in outs:
    if c is not None:
      c.wait()


def kernel(users, items, user_table, item_table):
  b = users.shape[0]
  d = user_table.shape[1]
  n_chunks = b // (NUM_WORKERS * CHUNK)

  mesh = plsc.VectorSubcoreMesh(core_axis_name="c", subcore_axis_name="s")
  run = functools.partial(
      pl.kernel,
      mesh=mesh,
      out_type=jax.ShapeDtypeStruct((b, d), jnp.float32),
      scratch_types=[
          pltpu.VMEM((n_chunks * CHUNK,), jnp.int32),
          pltpu.VMEM((n_chunks * CHUNK,), jnp.int32),
          pltpu.VMEM((RING, CHUNK, d), jnp.float32),
          pltpu.VMEM((RING, CHUNK, d), jnp.float32),
          pltpu.SemaphoreType.DMA,
          pltpu.SemaphoreType.DMA,
          pltpu.SemaphoreType.DMA,
      ],
  )(_gmf_body)
  return run(users.astype(jnp.int32), items.astype(jnp.int32),
             user_table, item_table)


# speedup vs baseline: 1.2756x; 1.0054x over previous
import functools

import jax
import jax.numpy as jnp
from jax import lax
from jax.experimental import pallas as pl
from jax.experimental.pallas import tpu as pltpu
from jax.experimental.pallas import tpu_sc as plsc

L = 16
NUM_WORKERS = 32
CHUNK = 128
RING = 3


def _gmf_body(users_hbm, items_hbm, ut_hbm, it_hbm, out_hbm,
              idx_u, idx_v, rows_u, rows_v, sem_u, sem_v, sem_o):
  n_chunks = idx_u.shape[0] // CHUNK
  d = ut_hbm.shape[1]
  wid = lax.axis_index("s") * 2 + lax.axis_index("c")
  base = wid * n_chunks * CHUNK

  ci = pltpu.async_copy(users_hbm.at[pl.ds(base, n_chunks * CHUNK)], idx_u,
                        sem_u)
  cj = pltpu.async_copy(items_hbm.at[pl.ds(base, n_chunks * CHUNK)], idx_v,
                        sem_v)
  ci.wait()
  cj.wait()

  def fire(j):
    slot = j % RING
    cu = pltpu.async_copy(ut_hbm.at[idx_u.at[pl.ds(j * CHUNK, CHUNK)]],
                          rows_u.at[slot], sem_u)
    cv = pltpu.async_copy(it_hbm.at[idx_v.at[pl.ds(j * CHUNK, CHUNK)]],
                          rows_v.at[slot], sem_v)
    return cu, cv

  gathers = [fire(j) for j in range(min(RING, n_chunks))]
  gathers += [None] * (n_chunks - len(gathers))
  outs = [None] * n_chunks

  for j in range(n_chunks):
    if RING <= n_chunks and 1 <= j <= n_chunks - RING:
      outs[j - 1].wait()
      outs[j - 1] = None
      gathers[j + RING - 1] = fire(j + RING - 1)

    cu, cv = gathers[j]
    cu.wait()
    cv.wait()
    slot = j % RING

    def mul_row(r, _):
      for k in range(d // L):
        s = pl.ds(k * L, L)
        rows_u[slot, r, s] = rows_u[slot, r, s] * rows_v[slot, r, s]
      return _

    lax.fori_loop(0, CHUNK, mul_row, 0, unroll=4)
    outs[j] = pltpu.async_copy(
        rows_u.at[slot],
        out_hbm.at[pl.ds((wid * n_chunks + j) * CHUNK, CHUNK)], sem_o)

  for c in outs:
    if c is not None:
      c.wait()


def kernel(users, items, user_table, item_table):
  b = users.shape[0]
  d = user_table.shape[1]
  n_chunks = b // (NUM_WORKERS * CHUNK)

  mesh = plsc.VectorSubcoreMesh(core_axis_name="c", subcore_axis_name="s")
  run = functools.partial(
      pl.kernel,
      mesh=mesh,
      out_type=jax.ShapeDtypeStruct((b, d), jnp.float32),
      scratch_types=[
          pltpu.VMEM((n_chunks * CHUNK,), jnp.int32),
          pltpu.VMEM((n_chunks * CHUNK,), jnp.int32),
          pltpu.VMEM((RING, CHUNK, d), jnp.float32),
          pltpu.VMEM((RING, CHUNK, d), jnp.float32),
          pltpu.SemaphoreType.DMA,
          pltpu.SemaphoreType.DMA,
          pltpu.SemaphoreType.DMA,
      ],
  )(_gmf_body)
  return run(users.astype(jnp.int32), items.astype(jnp.int32),
             user_table, item_table)


# speedup vs baseline: 1.2758x; 1.0002x over previous
import functools

import jax
import jax.numpy as jnp
from jax import lax
from jax.experimental import pallas as pl
from jax.experimental.pallas import tpu as pltpu
from jax.experimental.pallas import tpu_sc as plsc

L = 16
NUM_WORKERS = 32
CHUNK = 128
RING = 3


def _gmf_body(users_hbm, items_hbm, ut_hbm, it_hbm, out_hbm,
              idx_u, idx_v, rows_u, rows_v, sem_u, sem_v, sem_o):
  n_chunks = idx_u.shape[0] // CHUNK
  d = ut_hbm.shape[1]
  wid = lax.axis_index("s") * 2 + lax.axis_index("c")
  base = wid * n_chunks * CHUNK

  rest = (n_chunks - 1) * CHUNK
  ci0 = pltpu.async_copy(users_hbm.at[pl.ds(base, CHUNK)],
                         idx_u.at[pl.ds(0, CHUNK)], sem_u)
  cj0 = pltpu.async_copy(items_hbm.at[pl.ds(base, CHUNK)],
                         idx_v.at[pl.ds(0, CHUNK)], sem_v)
  ci1 = pltpu.async_copy(users_hbm.at[pl.ds(base + CHUNK, rest)],
                         idx_u.at[pl.ds(CHUNK, rest)], sem_u)
  cj1 = pltpu.async_copy(items_hbm.at[pl.ds(base + CHUNK, rest)],
                         idx_v.at[pl.ds(CHUNK, rest)], sem_v)
  ci0.wait()
  cj0.wait()

  def fire(j):
    slot = j % RING
    cu = pltpu.async_copy(ut_hbm.at[idx_u.at[pl.ds(j * CHUNK, CHUNK)]],
                          rows_u.at[slot], sem_u)
    cv = pltpu.async_copy(it_hbm.at[idx_v.at[pl.ds(j * CHUNK, CHUNK)]],
                          rows_v.at[slot], sem_v)
    return cu, cv

  gathers = [fire(0)]
  ci1.wait()
  cj1.wait()
  gathers += [fire(j) for j in range(1, min(RING, n_chunks))]
  gathers += [None] * (n_chunks - len(gathers))
  outs = [None] * n_chunks

  for j in range(n_chunks):
    if RING <= n_chunks and 1 <= j <= n_chunks - RING:
      outs[j - 1].wait()
      outs[j - 1] = None
      gathers[j + RING - 1] = fire(j + RING - 1)

    cu, cv = gathers[j]
    cu.wait()
    cv.wait()
    slot = j % RING

    def mul_row(r, _):
      for k in range(d // L):
        s = pl.ds(k * L, L)
        rows_u[slot, r, s] = rows_u[slot, r, s] * rows_v[slot, r, s]
      return _

    lax.fori_loop(0, CHUNK, mul_row, 0, unroll=4)
    outs[j] = pltpu.async_copy(
        rows_u.at[slot],
        out_hbm.at[pl.ds((wid * n_chunks + j) * CHUNK, CHUNK)], sem_o)

  for c in outs:
    if c is not None:
      c.wait()


def kernel(users, items, user_table, item_table):
  b = users.shape[0]
  d = user_table.shape[1]
  n_chunks = b // (NUM_WORKERS * CHUNK)

  mesh = plsc.VectorSubcoreMesh(core_axis_name="c", subcore_axis_name="s")
  run = functools.partial(
      pl.kernel,
      mesh=mesh,
      out_type=jax.ShapeDtypeStruct((b, d), jnp.float32),
      scratch_types=[
          pltpu.VMEM((n_chunks * CHUNK,), jnp.int32),
          pltpu.VMEM((n_chunks * CHUNK,), jnp.int32),
          pltpu.VMEM((RING, CHUNK, d), jnp.float32),
          pltpu.VMEM((RING, CHUNK, d), jnp.float32),
          pltpu.SemaphoreType.DMA,
          pltpu.SemaphoreType.DMA,
          pltpu.SemaphoreType.DMA,
      ],
  )(_gmf_body)
  return run(users.astype(jnp.int32), items.astype(jnp.int32),
             user_table, item_table)
